# baseline (speedup 1.0000x reference)
import jax
import jax.numpy as jnp
from jax import lax
from jax.experimental import pallas as pl
from jax.experimental.pallas import tpu as pltpu
from jax.experimental.pallas import tpu_sc as plsc

BATCH = 4096
HIST = 50
D_DIM = 64

NC = 2
NS = 16
NW = NC * NS
B_TOTAL = BATCH * HIST
PER_W = B_TOTAL // NW
CHUNK = 128
NCHUNK = PER_W // CHUNK
NBUF = 5
N_OUTER = NCHUNK // NBUF


def _emb_body(x_hbm, table_hbm, out_hbm, idx_v, rows_v, gsem, osem):
    cid = lax.axis_index("c")
    sid = lax.axis_index("s")
    wid = sid * NC + cid
    base = wid * PER_W

    pltpu.sync_copy(x_hbm.at[wid], idx_v)

    def gather_start(gid, b):
        pltpu.make_async_copy(
            table_hbm.at[idx_v.at[gid]], rows_v.at[b], gsem.at[b]
        ).start()

    def gather_wait(b):
        pltpu.make_async_copy(
            table_hbm.at[idx_v.at[0]], rows_v.at[b], gsem.at[b]
        ).wait()

    def out_start(gid, b):
        pltpu.make_async_copy(
            rows_v.at[b], out_hbm.at[pl.ds(base + gid * CHUNK, CHUNK)], osem.at[b]
        ).start()

    def out_wait(b):
        pltpu.make_async_copy(
            rows_v.at[b], out_hbm.at[pl.ds(base, CHUNK)], osem.at[b]
        ).wait()

    for b in range(NBUF):
        gather_start(b, b)

    def outer(g, _):
        for b in range(NBUF):
            gid = g * NBUF + b
            gather_wait(b)
            out_start(gid, b)
            out_wait(b)

            @pl.when(gid + NBUF < NCHUNK)
            def _():
                gather_start(gid + NBUF, b)

        return ()

    lax.fori_loop(0, N_OUTER, outer, ())


def kernel(x, item_emb_mat):
    x_flat = x.reshape(NW, NCHUNK, CHUNK).astype(jnp.int32)
    mesh = plsc.VectorSubcoreMesh(core_axis_name="c", subcore_axis_name="s")
    out = pl.kernel(
        _emb_body,
        out_type=jax.ShapeDtypeStruct((B_TOTAL, D_DIM), jnp.float32),
        mesh=mesh,
        compiler_params=pltpu.CompilerParams(use_tc_tiling_on_sc=False),
        scratch_types=[
            pltpu.VMEM((NCHUNK, CHUNK), jnp.int32),
            pltpu.VMEM((NBUF, CHUNK, D_DIM), jnp.float32),
            pltpu.SemaphoreType.DMA((NBUF,)),
            pltpu.SemaphoreType.DMA((NBUF,)),
        ],
    )(x_flat, item_emb_mat)
    return out.reshape(BATCH, HIST, D_DIM)
